# baseline (speedup 1.0000x reference)
import jax
import jax.numpy as jnp
from jax.experimental import pallas as pl
from jax.experimental.pallas import tpu as pltpu


def _body(x_ref, emb_ref, w_ref, out_ref, rows_ref, sem):
    b = rows_ref.shape[0]
    copies = [
        pltpu.make_async_copy(
            emb_ref.at[pl.ds(x_ref[i], 1), :], rows_ref.at[pl.ds(i, 1), :], sem
        )
        for i in range(b)
    ]
    for c in copies:
        c.start()
    for c in copies:
        c.wait()
    h = rows_ref[...] * jnp.float32(1.5)
    logits = jnp.dot(h, w_ref[...], preferred_element_type=jnp.float32)
    m = jnp.max(logits, axis=1, keepdims=True)
    e = jnp.exp(logits - m)
    out_ref[...] = e / jnp.sum(e, axis=1, keepdims=True)


def kernel(x_init, emb, W, source_p):
    del source_p
    b = x_init.shape[0]
    v, d = emb.shape
    return pl.pallas_call(
        _body,
        in_specs=[
            pl.BlockSpec(memory_space=pltpu.SMEM),
            pl.BlockSpec(memory_space=pl.ANY),
            pl.BlockSpec(memory_space=pltpu.VMEM),
        ],
        out_specs=pl.BlockSpec(memory_space=pltpu.VMEM),
        scratch_shapes=[
            pltpu.VMEM((b, d), jnp.float32),
            pltpu.SemaphoreType.DMA,
        ],
        out_shape=jax.ShapeDtypeStruct((b, v), jnp.float32),
    )(x_init, emb, W)


# speedup vs baseline: 1.1534x; 1.1534x over previous
import jax
import jax.numpy as jnp
from jax.experimental import pallas as pl
from jax.experimental.pallas import tpu as pltpu


def _body(x_ref, emb_ref, w_ref, out_ref, rows_ref):
    b = rows_ref.shape[0]
    for i in range(b):
        rows_ref[i, :] = emb_ref[x_ref[i], :]
    h = rows_ref[...] * jnp.float32(1.5)
    logits = jnp.dot(h, w_ref[...], preferred_element_type=jnp.float32)
    m = jnp.max(logits, axis=1, keepdims=True)
    e = jnp.exp(logits - m)
    out_ref[...] = e / jnp.sum(e, axis=1, keepdims=True)


def kernel(x_init, emb, W, source_p):
    del source_p
    b = x_init.shape[0]
    v, d = emb.shape
    return pl.pallas_call(
        _body,
        compiler_params=pltpu.CompilerParams(
            disable_bounds_checks=True,
            disable_semaphore_checks=True,
            skip_device_barrier=True,
        ),
        in_specs=[
            pl.BlockSpec(memory_space=pltpu.SMEM),
            pl.BlockSpec(memory_space=pltpu.VMEM),
            pl.BlockSpec(memory_space=pltpu.VMEM),
        ],
        out_specs=pl.BlockSpec(memory_space=pltpu.VMEM),
        scratch_shapes=[pltpu.VMEM((b, d), jnp.float32)],
        out_shape=jax.ShapeDtypeStruct((b, v), jnp.float32),
    )(x_init, emb, W)
